# baseline (speedup 1.0000x reference)
import functools

import jax
import jax.numpy as jnp
from jax import lax
from jax.experimental import pallas as pl
from jax.experimental.pallas import tpu as pltpu
from jax.experimental.pallas import tpu_sc as plsc

B = 16384
D = 32
NC = 2
NS = 16
NW = NC * NS
BPW = B // NW
IDX_MINOR = 128
IDX_ROWS = BPW // IDX_MINOR
GROUPS = BPW // 16


def _body(uid_hbm, mid_hbm, ut_hbm, mt_hbm, fc_hbm, out_hbm,
          uidx_v, midx_v, urows_v, mrows_v, out_v, fc_v, sem):
    wid = lax.axis_index("s") * NC + lax.axis_index("c")
    base = wid * IDX_ROWS

    pltpu.sync_copy(uid_hbm.at[pl.ds(base, IDX_ROWS)], uidx_v)
    pltpu.sync_copy(mid_hbm.at[pl.ds(base, IDX_ROWS)], midx_v)
    pltpu.sync_copy(fc_hbm, fc_v)

    copies = []
    for i in range(IDX_ROWS):
        copies.append(pltpu.async_copy(
            ut_hbm.at[uidx_v.at[i]],
            urows_v.at[pl.ds(i * IDX_MINOR, IDX_MINOR)], sem))
        copies.append(pltpu.async_copy(
            mt_hbm.at[midx_v.at[i]],
            mrows_v.at[pl.ds(i * IDX_MINOR, IDX_MINOR)], sem))
    for cp in copies:
        cp.wait()

    w_vec = fc_v[0, :]
    b_vec = fc_v[1, :]

    lane = lax.iota(jnp.int32, 16)

    def group(g, carry):
        res = jnp.zeros((16,), jnp.float32)
        for r in range(16):
            i = g * 16 + r
            t = (urows_v[i, pl.ds(0, 16)] * mrows_v[i, pl.ds(0, 16)] +
                 urows_v[i, pl.ds(16, 16)] * mrows_v[i, pl.ds(16, 16)])
            res = jnp.where(lane == r, jnp.sum(t), res)
        y = res * w_vec + b_vec
        out_v[pl.ds(g * 16, 16)] = 1.0 / (1.0 + jnp.exp(-y))
        return carry

    lax.fori_loop(0, GROUPS, group, 0)

    pltpu.sync_copy(out_v, out_hbm.at[pl.ds(wid * BPW, BPW)])


@jax.jit
def _run(uid2, mid2, u_table, m_table, fc16):
    mesh = plsc.VectorSubcoreMesh(core_axis_name="c", subcore_axis_name="s")
    f = functools.partial(
        pl.kernel,
        mesh=mesh,
        compiler_params=pltpu.CompilerParams(needs_layout_passes=False,
                                             use_tc_tiling_on_sc=False),
        out_type=jax.ShapeDtypeStruct((B,), jnp.float32),
        scratch_types=[
            pltpu.VMEM((IDX_ROWS, IDX_MINOR), jnp.int32),
            pltpu.VMEM((IDX_ROWS, IDX_MINOR), jnp.int32),
            pltpu.VMEM((BPW, D), jnp.float32),
            pltpu.VMEM((BPW, D), jnp.float32),
            pltpu.VMEM((BPW,), jnp.float32),
            pltpu.VMEM((2, 16), jnp.float32),
            pltpu.SemaphoreType.DMA,
        ],
    )(_body)
    return f(uid2, mid2, u_table, m_table, fc16)


def kernel(user_ids, movie_ids, u_table, m_table, fc_w, fc_b):
    uid2 = user_ids.astype(jnp.int32).reshape(B // IDX_MINOR, IDX_MINOR)
    mid2 = movie_ids.astype(jnp.int32).reshape(B // IDX_MINOR, IDX_MINOR)
    fc16 = jnp.stack([jnp.full((16,), fc_w.reshape(()), jnp.float32),
                      jnp.full((16,), fc_b.reshape(()), jnp.float32)])
    out = _run(uid2, mid2, u_table, m_table, fc16)
    return out.reshape(B, 1)
